# baseline (speedup 1.0000x reference)
import functools

import jax
import jax.numpy as jnp
from jax import lax
from jax.experimental import pallas as pl
from jax.experimental.pallas import tpu as pltpu
from jax.experimental.pallas import tpu_sc as plsc

_M, _B, _D = 100000, 16384, 128
_NW = 32
_CPW = _D // _NW
_CHUNK = 4096
_TR_BLK = 4096
_ADD_BLK = 4096


def _tc_transpose2(index, src):
    def body(idx_ref, src_ref, oidx_ref, osrc_ref):
        oidx_ref[...] = idx_ref[...].T
        osrc_ref[...] = src_ref[...].T

    return pl.pallas_call(
        body,
        grid=(_B // _TR_BLK,),
        in_specs=[
            pl.BlockSpec((_TR_BLK, _D), lambda i: (i, 0)),
            pl.BlockSpec((_TR_BLK, _D), lambda i: (i, 0)),
        ],
        out_specs=[
            pl.BlockSpec((_D, _TR_BLK), lambda i: (0, i)),
            pl.BlockSpec((_D, _TR_BLK), lambda i: (0, i)),
        ],
        out_shape=[
            jax.ShapeDtypeStruct((_D, _B), jnp.int32),
            jax.ShapeDtypeStruct((_D, _B), jnp.float32),
        ],
    )(index, src)


_NCH = _B // _CHUNK
_MP = 100096
_QOFF = (0, 25600, 51200, 76800)
_QLEN = (25600, 25600, 25600, 23296)


def _scatter_cols(idxT, srcT):
    mesh = plsc.VectorSubcoreMesh(core_axis_name="c", subcore_axis_name="s")

    @functools.partial(
        pl.kernel,
        out_type=jax.ShapeDtypeStruct((_D, _MP), jnp.float32),
        mesh=mesh,
        scratch_types=[
            pltpu.VMEM((_MP,), jnp.float32),
            pltpu.VMEM((_CHUNK,), jnp.int32),
            pltpu.VMEM((_CHUNK,), jnp.int32),
            pltpu.VMEM((_CHUNK,), jnp.float32),
            pltpu.VMEM((_CHUNK,), jnp.float32),
            pltpu.VMEM_SHARED((_MP,), jnp.float32),
            [pltpu.SemaphoreType.DMA] * 4,
            [pltpu.SemaphoreType.DMA] * 4,
            [pltpu.SemaphoreType.DMA] * 4,
        ],
        compiler_params=pltpu.CompilerParams(needs_layout_passes=False),
    )
    def k(idxT_hbm, srcT_hbm, out_hbm, buf, idxv0, idxv1, srcv0, srcv1, zsh,
          osems, isems, zsems):
        idxv = (idxv0, idxv1)
        srcv = (srcv0, srcv1)
        wid = lax.axis_index("s") * 2 + lax.axis_index("c")
        sid = lax.axis_index("s")
        zeros = jnp.zeros((16,), jnp.float32)

        ZU = 16
        SU = 16

        ZSL = _MP // 16

        def zsetup(i, c):
            base = i * (17 * 16)
            for u in range(17):
                buf[pl.ds(base + u * 16, 16)] = zeros
            return c

        lax.fori_loop(0, ZSL // (17 * 16), zsetup, 0)
        pltpu.sync_copy(buf.at[pl.ds(0, ZSL)], zsh.at[pl.ds(sid * ZSL, ZSL)])
        plsc.subcore_barrier()

        def start_in(j, ch):
            s = ch % 2
            hi = pltpu.async_copy(
                idxT_hbm.at[j, pl.ds(ch * _CHUNK, _CHUNK)], idxv[s], isems[2 * s]
            )
            hs = pltpu.async_copy(
                srcT_hbm.at[j, pl.ds(ch * _CHUNK, _CHUNK)], srcv[s], isems[2 * s + 1]
            )
            return (hi, hs)

        out_handles = [None] * 4

        for cc in range(_CPW):
            j = wid * _CPW + cc
            in_handles = [start_in(j, 0), start_in(j, 1), None, None]

            zin_handles = []
            for q in range(4):
                if out_handles[q] is not None:
                    out_handles[q].wait()
                zin_handles.append(
                    pltpu.async_copy(
                        zsh.at[pl.ds(_QOFF[q], _QLEN[q])],
                        buf.at[pl.ds(_QOFF[q], _QLEN[q])],
                        zsems[q],
                    )
                )
            for h in zin_handles:
                h.wait()

            for ch in range(_NCH):
                s = ch % 2
                hi, hs = in_handles[ch]
                hi.wait()
                hs.wait()

                def sbody(i, c2, _s=s):
                    base = i * (SU * 16)
                    for u in range(SU):
                        vi = idxv[_s][pl.ds(base + u * 16, 16)]
                        vv = srcv[_s][pl.ds(base + u * 16, 16)]
                        plsc.addupdate_scatter(buf, [vi], vv)
                    return c2

                lax.fori_loop(0, _CHUNK // (SU * 16), sbody, 0)
                if ch + 2 < _NCH:
                    in_handles[ch + 2] = start_in(j, ch + 2)

            for q in range(4):
                out_handles[q] = pltpu.async_copy(
                    buf.at[pl.ds(_QOFF[q], _QLEN[q])],
                    out_hbm.at[j, pl.ds(_QOFF[q], _QLEN[q])],
                    osems[q],
                )

        for q in range(4):
            out_handles[q].wait()

    return k(idxT, srcT)


def _tc_add_t(input, delta):
    def body(in_ref, d_ref, o_ref):
        o_ref[...] = in_ref[...] + d_ref[...].T

    grid = (_M + _ADD_BLK - 1) // _ADD_BLK
    return pl.pallas_call(
        body,
        grid=(grid,),
        in_specs=[
            pl.BlockSpec((_ADD_BLK, _D), lambda i: (i, 0)),
            pl.BlockSpec((_D, _ADD_BLK), lambda i: (0, i)),
        ],
        out_specs=pl.BlockSpec((_ADD_BLK, _D), lambda i: (i, 0)),
        out_shape=jax.ShapeDtypeStruct((_M, _D), jnp.float32),
    )(input, delta)


@jax.jit
def kernel(input, index, src):
    idxT, srcT = _tc_transpose2(index, src)
    delta = _scatter_cols(idxT, srcT)
    return _tc_add_t(input, delta)


# speedup vs baseline: 1.0410x; 1.0410x over previous
import functools

import jax
import jax.numpy as jnp
from jax import lax
from jax.experimental import pallas as pl
from jax.experimental.pallas import tpu as pltpu
from jax.experimental.pallas import tpu_sc as plsc

_M, _B, _D = 100000, 16384, 128
_NW = 32
_CPW = _D // _NW
_CHUNK = 4096
_TR_BLK = 8192
_ADD_BLK = 8192


def _tc_transpose2(index, src):
    def body(idx_ref, src_ref, oidx_ref, osrc_ref):
        oidx_ref[...] = idx_ref[...].T
        osrc_ref[...] = src_ref[...].T

    return pl.pallas_call(
        body,
        grid=(_B // _TR_BLK,),
        in_specs=[
            pl.BlockSpec((_TR_BLK, _D), lambda i: (i, 0)),
            pl.BlockSpec((_TR_BLK, _D), lambda i: (i, 0)),
        ],
        out_specs=[
            pl.BlockSpec((_D, _TR_BLK), lambda i: (0, i)),
            pl.BlockSpec((_D, _TR_BLK), lambda i: (0, i)),
        ],
        out_shape=[
            jax.ShapeDtypeStruct((_D, _B), jnp.int32),
            jax.ShapeDtypeStruct((_D, _B), jnp.float32),
        ],
    )(index, src)


_NCH = _B // _CHUNK
_MP = 100096
_QOFF = (0, 25600, 51200, 76800)
_QLEN = (25600, 25600, 25600, 23296)


def _scatter_cols(idxT, srcT):
    mesh = plsc.VectorSubcoreMesh(core_axis_name="c", subcore_axis_name="s")

    @functools.partial(
        pl.kernel,
        out_type=jax.ShapeDtypeStruct((_D, _MP), jnp.float32),
        mesh=mesh,
        scratch_types=[
            pltpu.VMEM((_MP,), jnp.float32),
            pltpu.VMEM((_CHUNK,), jnp.int32),
            pltpu.VMEM((_CHUNK,), jnp.int32),
            pltpu.VMEM((_CHUNK,), jnp.float32),
            pltpu.VMEM((_CHUNK,), jnp.float32),
            [pltpu.SemaphoreType.DMA] * 4,
            [pltpu.SemaphoreType.DMA] * 4,
        ],
        compiler_params=pltpu.CompilerParams(needs_layout_passes=False),
    )
    def k(idxT_hbm, srcT_hbm, out_hbm, buf, idxv0, idxv1, srcv0, srcv1,
          osems, isems):
        idxv = (idxv0, idxv1)
        srcv = (srcv0, srcv1)
        wid = lax.axis_index("s") * 2 + lax.axis_index("c")
        zeros = jnp.zeros((16,), jnp.float32)

        ZU = 16
        SU = 16

        def start_in(j, ch):
            s = ch % 2
            hi = pltpu.async_copy(
                idxT_hbm.at[j, pl.ds(ch * _CHUNK, _CHUNK)], idxv[s], isems[2 * s]
            )
            hs = pltpu.async_copy(
                srcT_hbm.at[j, pl.ds(ch * _CHUNK, _CHUNK)], srcv[s], isems[2 * s + 1]
            )
            return (hi, hs)

        out_handles = [None] * 4

        for cc in range(_CPW):
            j = wid * _CPW + cc
            in_handles = [start_in(j, 0), start_in(j, 1), None, None]

            for q in range(4):
                if out_handles[q] is not None:
                    out_handles[q].wait()

                def zbody(i, c, _qo=_QOFF[q]):
                    base = _qo + i * (ZU * 16)
                    for u in range(ZU):
                        buf[pl.ds(base + u * 16, 16)] = zeros
                    return c

                lax.fori_loop(0, _QLEN[q] // (ZU * 16), zbody, 0)

            for ch in range(_NCH):
                s = ch % 2
                hi, hs = in_handles[ch]
                hi.wait()
                hs.wait()

                def sbody(i, c2, _s=s):
                    base = i * (SU * 16)
                    for u in range(SU):
                        vi = idxv[_s][pl.ds(base + u * 16, 16)]
                        vv = srcv[_s][pl.ds(base + u * 16, 16)]
                        plsc.addupdate_scatter(buf, [vi], vv)
                    return c2

                lax.fori_loop(0, _CHUNK // (SU * 16), sbody, 0)
                if ch + 2 < _NCH:
                    in_handles[ch + 2] = start_in(j, ch + 2)

            for q in range(4):
                out_handles[q] = pltpu.async_copy(
                    buf.at[pl.ds(_QOFF[q], _QLEN[q])],
                    out_hbm.at[j, pl.ds(_QOFF[q], _QLEN[q])],
                    osems[q],
                )

        for q in range(4):
            out_handles[q].wait()

    return k(idxT, srcT)


def _tc_add_t(input, delta):
    def body(in_ref, d_ref, o_ref):
        o_ref[...] = in_ref[...] + d_ref[...].T

    grid = (_M + _ADD_BLK - 1) // _ADD_BLK
    return pl.pallas_call(
        body,
        grid=(grid,),
        in_specs=[
            pl.BlockSpec((_ADD_BLK, _D), lambda i: (i, 0)),
            pl.BlockSpec((_D, _ADD_BLK), lambda i: (0, i)),
        ],
        out_specs=pl.BlockSpec((_ADD_BLK, _D), lambda i: (i, 0)),
        out_shape=jax.ShapeDtypeStruct((_M, _D), jnp.float32),
    )(input, delta)


@jax.jit
def kernel(input, index, src):
    idxT, srcT = _tc_transpose2(index, src)
    delta = _scatter_cols(idxT, srcT)
    return _tc_add_t(input, delta)


# speedup vs baseline: 1.0837x; 1.0409x over previous
import functools

import jax
import jax.numpy as jnp
from jax import lax
from jax.experimental import pallas as pl
from jax.experimental.pallas import tpu as pltpu
from jax.experimental.pallas import tpu_sc as plsc

_M, _B, _D = 100000, 16384, 128
_NW = 32
_CPW = _D // _NW
_CHUNK = 4096
_TR_BLK = 8192
_ADD_BLK = 8192


def _tc_transpose2(index, src):
    def body(idx_ref, src_ref, oidx_ref, osrc_ref):
        oidx_ref[...] = idx_ref[...].T
        osrc_ref[...] = src_ref[...].T

    return pl.pallas_call(
        body,
        grid=(_B // _TR_BLK,),
        in_specs=[
            pl.BlockSpec((_TR_BLK, _D), lambda i: (i, 0)),
            pl.BlockSpec((_TR_BLK, _D), lambda i: (i, 0)),
        ],
        out_specs=[
            pl.BlockSpec((_D, _TR_BLK), lambda i: (0, i)),
            pl.BlockSpec((_D, _TR_BLK), lambda i: (0, i)),
        ],
        out_shape=[
            jax.ShapeDtypeStruct((_D, _B), jnp.int32),
            jax.ShapeDtypeStruct((_D, _B), jnp.float32),
        ],
    )(index, src)


_NCH = _B // _CHUNK
_MP = 100096
_QOFF = (0, 12288, 24576, 36864, 49152, 61440, 73728, 86016)
_QLEN = (12288, 12288, 12288, 12288, 12288, 12288, 12288, 14080)
_NQ = len(_QOFF)


def _scatter_cols(idxT, srcT):
    mesh = plsc.VectorSubcoreMesh(core_axis_name="c", subcore_axis_name="s")

    @functools.partial(
        pl.kernel,
        out_type=jax.ShapeDtypeStruct((_D, _MP), jnp.float32),
        mesh=mesh,
        scratch_types=[
            pltpu.VMEM((_MP,), jnp.float32),
            pltpu.VMEM((_CHUNK,), jnp.int32),
            pltpu.VMEM((_CHUNK,), jnp.int32),
            pltpu.VMEM((_CHUNK,), jnp.float32),
            pltpu.VMEM((_CHUNK,), jnp.float32),
            [pltpu.SemaphoreType.DMA] * _NQ,
            [pltpu.SemaphoreType.DMA] * 4,
        ],
        compiler_params=pltpu.CompilerParams(needs_layout_passes=False),
    )
    def k(idxT_hbm, srcT_hbm, out_hbm, buf, idxv0, idxv1, srcv0, srcv1,
          osems, isems):
        idxv = (idxv0, idxv1)
        srcv = (srcv0, srcv1)
        wid = lax.axis_index("s") * 2 + lax.axis_index("c")
        zeros = jnp.zeros((16,), jnp.float32)

        ZU = 16
        SU = 16

        def start_in(j, ch):
            s = ch % 2
            hi = pltpu.async_copy(
                idxT_hbm.at[j, pl.ds(ch * _CHUNK, _CHUNK)], idxv[s], isems[2 * s]
            )
            hs = pltpu.async_copy(
                srcT_hbm.at[j, pl.ds(ch * _CHUNK, _CHUNK)], srcv[s], isems[2 * s + 1]
            )
            return (hi, hs)

        out_handles = [None] * _NQ

        for cc in range(_CPW):
            j = wid * _CPW + cc
            in_handles = [start_in(j, 0), start_in(j, 1), None, None]

            for q in range(_NQ):
                if out_handles[q] is not None:
                    out_handles[q].wait()

                def zbody(i, c, _qo=_QOFF[q]):
                    base = _qo + i * (ZU * 16)
                    for u in range(ZU):
                        buf[pl.ds(base + u * 16, 16)] = zeros
                    return c

                lax.fori_loop(0, _QLEN[q] // (ZU * 16), zbody, 0)

            for ch in range(_NCH):
                s = ch % 2
                hi, hs = in_handles[ch]
                hi.wait()
                hs.wait()

                def sbody(i, c2, _s=s):
                    base = i * (SU * 16)
                    for u in range(SU):
                        vi = idxv[_s][pl.ds(base + u * 16, 16)]
                        vv = srcv[_s][pl.ds(base + u * 16, 16)]
                        plsc.addupdate_scatter(buf, [vi], vv)
                    return c2

                lax.fori_loop(0, _CHUNK // (SU * 16), sbody, 0)
                if ch + 2 < _NCH:
                    in_handles[ch + 2] = start_in(j, ch + 2)

            for q in range(_NQ):
                out_handles[q] = pltpu.async_copy(
                    buf.at[pl.ds(_QOFF[q], _QLEN[q])],
                    out_hbm.at[j, pl.ds(_QOFF[q], _QLEN[q])],
                    osems[q],
                )

        for q in range(_NQ):
            out_handles[q].wait()

    return k(idxT, srcT)


def _tc_add_t(input, delta):
    def body(in_ref, d_ref, o_ref):
        o_ref[...] = in_ref[...] + d_ref[...].T

    grid = (_M + _ADD_BLK - 1) // _ADD_BLK
    return pl.pallas_call(
        body,
        grid=(grid,),
        in_specs=[
            pl.BlockSpec((_ADD_BLK, _D), lambda i: (i, 0)),
            pl.BlockSpec((_D, _ADD_BLK), lambda i: (0, i)),
        ],
        out_specs=pl.BlockSpec((_ADD_BLK, _D), lambda i: (i, 0)),
        out_shape=jax.ShapeDtypeStruct((_M, _D), jnp.float32),
    )(input, delta)


@jax.jit
def kernel(input, index, src):
    idxT, srcT = _tc_transpose2(index, src)
    delta = _scatter_cols(idxT, srcT)
    return _tc_add_t(input, delta)
